# baseline (speedup 1.0000x reference)
import jax
import jax.numpy as jnp
from jax import lax
from jax.experimental import pallas as pl
from jax.experimental.pallas import tpu as pltpu
from jax.experimental.pallas import tpu_sc as plsc

NENT = 10000
NSEG = 10240
CH = 128
NEDGE = 320000
NRELROW = 9
NC = 2
NS = 16
NW = NC * NS
EPW = 10240
EPAD = NW * EPW
K = 80
KH = K // 2
NCHUNK = EPW // K
NG = K // 16
BLK = 8
BLKE = BLK * K
NBLOCK = NCHUNK // BLK
DEPTH = 4
ROWS_PER_TILE = NSEG // NS
MBLK = 128
MROUNDS = NSEG // MBLK

_MESH = plsc.VectorSubcoreMesh(core_axis_name="c", subcore_axis_name="s")
_f32 = jnp.float32
_i32 = jnp.int32


def _c(v):
    return jnp.array(v, _i32)


def _worker_id():
    return lax.axis_index("s") * _c(NC) + lax.axis_index("c")


def _retry_scatter_max(tab, hidx, sv):
    def bdy(go):
        cur = plsc.load_gather(tab, [hidx])
        plsc.store_scatter(tab, [hidx], jnp.maximum(sv, cur), mask=sv > cur)
        chk = plsc.load_gather(tab, [hidx])
        return jnp.max((sv > chk).astype(_i32))
    lax.while_loop(lambda go: go > _c(0), bdy, _c(1))


def _gather_rows_split(table_hbm, idxb, rows, sem):
    d0 = pltpu.async_copy(table_hbm.at[idxb.at[pl.ds(0, KH)]],
                          rows.at[pl.ds(0, KH)], sem)
    d1 = pltpu.async_copy(table_hbm.at[idxb.at[pl.ds(KH, KH)]],
                          rows.at[pl.ds(KH, KH)], sem)
    return (d0, d1)


def _sca_body(a_hbm, x_hbm, head_hbm, rel_hbm, tail_hbm,
              scores_hbm, maxpart_hbm,
              headblk, relblk, tailblk, sball, maxtab,
              arows0, arows1, arows2, arows3,
              trows0, trows1, trows2, trows3,
              cidxb0, cidxb1, cidxb2, cidxb3,
              tailb0, tailb1, tailb2, tailb3,
              sem0, sem1, sem2, sem3):
    w = _worker_id()
    base0 = w * _c(EPW)
    lanes = lax.iota(_i32, 16)
    zero16 = jnp.zeros((16,), _f32)
    zero16i = jnp.zeros((16,), _i32)

    def initb(i, carry):
        plsc.store_scatter(maxtab, [lanes + i * _c(16)], zero16)
        return carry
    lax.fori_loop(_c(0), _c(NSEG // 16), initb, _c(0))

    arows = (arows0, arows1, arows2, arows3)
    trows = (trows0, trows1, trows2, trows3)
    cidxb = (cidxb0, cidxb1, cidxb2, cidxb3)
    tailb = (tailb0, tailb1, tailb2, tailb3)
    sems = (sem0, sem1, sem2, sem3)

    def issue(j, slot):
        for g in range(NG):
            off = j * K + g * 16
            h16 = headblk[pl.ds(off, 16)]
            r16 = relblk[pl.ds(off, 16)]
            rid = jnp.where(r16 == _c(0), _c(NRELROW - 1), r16 - _c(1))
            cidxb[slot][pl.ds(g * 16, 16)] = rid * _c(NSEG) + h16
            tailb[slot][pl.ds(g * 16, 16)] = tailblk[pl.ds(off, 16)]
        da = _gather_rows_split(a_hbm, cidxb[slot], arows[slot], sems[slot])
        dt = _gather_rows_split(x_hbm, tailb[slot], trows[slot], sems[slot])
        return da + dt

    def block(b, carry):
        eb0 = base0 + b * _c(BLKE)
        pltpu.sync_copy(head_hbm.at[pl.ds(eb0, BLKE)], headblk)
        pltpu.sync_copy(rel_hbm.at[pl.ds(eb0, BLKE)], relblk)
        pltpu.sync_copy(tail_hbm.at[pl.ds(eb0, BLKE)], tailblk)
        ds_ = [None] * BLK
        for p in range(DEPTH):
            ds_[p] = issue(p, p)
        for j in range(BLK):
            sl = j % DEPTH
            if j + DEPTH < BLK:
                ds_[j + DEPTH] = issue(j + DEPTH, sl)
            for d in ds_[j]:
                d.wait()
            for g in range(NG):
                eidx = lanes + _c(g * 16)

                def cbody(i2, accs, sl=sl, eidx=eidx):
                    ch0 = i2 * _c(8)
                    outs = list(accs)
                    for k in range(8):
                        col = zero16i + (ch0 + _c(k))
                        av = plsc.load_gather(arows[sl], [eidx, col])
                        tv = plsc.load_gather(trows[sl], [eidx, col])
                        outs[k % 4] = outs[k % 4] + av * tv
                    return tuple(outs)
                accs = lax.fori_loop(_c(0), _c(CH // 8), cbody,
                                     (zero16, zero16, zero16, zero16))
                dot = (accs[0] + accs[1]) + (accs[2] + accs[3])
                sv = jnp.exp(dot)
                gidx = lanes + b * _c(BLKE) + _c(j * K + g * 16)
                plsc.store_scatter(sball, [gidx], sv)
                hidx = headblk[pl.ds(j * K + g * 16, 16)]
                _retry_scatter_max(maxtab, hidx, sv)
        return carry
    lax.fori_loop(_c(0), _c(NBLOCK), block, _c(0))

    pltpu.sync_copy(sball, scores_hbm.at[pl.ds(base0, EPW)])
    pltpu.sync_copy(maxtab, maxpart_hbm.at[w])


_sca = pl.kernel(
    _sca_body,
    out_type=[jax.ShapeDtypeStruct((EPAD,), _f32),
              jax.ShapeDtypeStruct((NW, NSEG), _f32)],
    mesh=_MESH,
    compiler_params=pltpu.CompilerParams(needs_layout_passes=False),
    scratch_types=(
        [pltpu.VMEM((BLKE,), _i32)] * 3
        + [pltpu.VMEM((EPW,), _f32),
           pltpu.VMEM((NSEG,), _f32)]
        + [pltpu.VMEM((K, CH), _f32)] * 8
        + [pltpu.VMEM((K,), _i32)] * 8
        + [pltpu.SemaphoreType.DMA] * 4
    ),
)


def _scm_body(head_hbm, scores_hbm, maxpart_hbm,
              evals_hbm,
              headall, sball, eall, mtab, mstage):
    w = _worker_id()
    base0 = w * _c(EPW)
    lanes = lax.iota(_i32, 16)

    pltpu.sync_copy(head_hbm.at[pl.ds(base0, EPW)], headall)
    pltpu.sync_copy(scores_hbm.at[pl.ds(base0, EPW)], sball)

    def mround(r, carry):
        seg0 = r * _c(MBLK)
        pltpu.sync_copy(maxpart_hbm.at[:, pl.ds(seg0, MBLK)], mstage)
        for g in range(MBLK // 16):
            idx = lanes + _c(g * 16)
            m = plsc.load_gather(mstage, [jnp.full((16,), 0, _i32), idx])
            for j in range(1, NW):
                vj = plsc.load_gather(mstage, [jnp.full((16,), j, _i32), idx])
                m = jnp.maximum(m, vj)
            plsc.store_scatter(mtab, [idx + seg0], m)
        return carry
    lax.fori_loop(_c(0), _c(MROUNDS), mround, _c(0))

    def ebody(i, carry):
        idx = lanes + i * _c(16)
        h16 = plsc.load_gather(headall, [idx])
        m16 = plsc.load_gather(mtab, [h16])
        s16 = plsc.load_gather(sball, [idx])
        plsc.store_scatter(eall, [idx], jnp.exp(s16 - m16))
        return carry
    lax.fori_loop(_c(0), _c(EPW // 16), ebody, _c(0))

    pltpu.sync_copy(eall, evals_hbm.at[pl.ds(base0, EPW)])


_scm = pl.kernel(
    _scm_body,
    out_type=[jax.ShapeDtypeStruct((EPAD,), _f32)],
    mesh=_MESH,
    compiler_params=pltpu.CompilerParams(needs_layout_passes=False),
    scratch_types=[
        pltpu.VMEM((EPW,), _i32),
        pltpu.VMEM((EPW,), _f32),
        pltpu.VMEM((EPW,), _f32),
        pltpu.VMEM((NSEG,), _f32),
        pltpu.VMEM((NW, MBLK), _f32),
    ],
)


def _zero_rows_buf(buf):
    lanes = lax.iota(_i32, 16)
    zero16 = jnp.zeros((16,), _f32)

    def zb(i, carry):
        flat = lanes + i * _c(16)
        plsc.store_scatter(buf, [lax.shift_right_logical(flat, _c(7)),
                                 lax.bitwise_and(flat, _c(127))], zero16)
        return carry
    lax.fori_loop(_c(0), _c(K * CH // 16), zb, _c(0))


def _zero_spmem_slice(spmem, buf, sid):
    def zloop(j, carry):
        pltpu.sync_copy(
            buf, spmem.at[pl.ds(sid * _c(ROWS_PER_TILE) + j * _c(K), K)])
        return carry
    lax.fori_loop(_c(0), _c(ROWS_PER_TILE // K), zloop, _c(0))


def _dump_spmem(spmem, buf, ypart_hbm, c, sid):
    for j in range(ROWS_PER_TILE // K):
        row = sid * _c(ROWS_PER_TILE) + _c(j * K)
        pltpu.sync_copy(spmem.at[pl.ds(row, K)], buf)
        pltpu.sync_copy(buf, ypart_hbm.at[c, pl.ds(row, K)])


def _scagg_body(x_hbm, idx_hbm, tail_hbm, ev_hbm,
                ypart_hbm,
                idxiblk, idxtblk, evblk,
                idxb0, idxb1, idxb2, idxb3,
                rows0, rows1, rows2, rows3,
                spmem, sem0, sem1, sem2, sem3):
    c = lax.axis_index("c")
    sid = lax.axis_index("s")
    w = _worker_id()
    base0 = w * _c(EPW)
    lanes = lax.iota(_i32, 16)

    _zero_rows_buf(rows0)
    _zero_spmem_slice(spmem, rows0, sid)
    plsc.subcore_barrier()

    rows = (rows0, rows1, rows2, rows3)
    idxb = (idxb0, idxb1, idxb2, idxb3)
    sems = (sem0, sem1, sem2, sem3)

    def block(b, carry):
        eb0 = base0 + b * _c(BLKE)
        pltpu.sync_copy(idx_hbm.at[pl.ds(eb0, BLKE)], idxiblk)
        pltpu.sync_copy(tail_hbm.at[pl.ds(eb0, BLKE)], idxtblk)
        pltpu.sync_copy(ev_hbm.at[pl.ds(eb0, BLKE)], evblk)
        ds_ = [None] * BLK

        def issue(j, slot):
            return _gather_rows_split(
                x_hbm, idxtblk.at[pl.ds(j * K, K)], rows[slot], sems[slot])
        for p in range(DEPTH):
            ds_[p] = issue(p, p)
        for j in range(BLK):
            sl = j % DEPTH
            if j + DEPTH < BLK:
                ds_[j + DEPTH] = issue(j + DEPTH, sl)
            for d in ds_[j]:
                d.wait()
            for g in range(NG):
                iv = idxiblk[pl.ds(j * K + g * 16, 16)]
                idxb[sl][pl.ds(g * 16, 16)] = iv
                ev = evblk[pl.ds(j * K + g * 16, 16)]
                eidx = lanes + _c(g * 16)

                def sbody(i2, carry2, sl=sl, eidx=eidx, ev=ev):
                    ch0 = i2 * _c(8)
                    for k in range(8):
                        col = jnp.zeros((16,), _i32) + (ch0 + _c(k))
                        tv = plsc.load_gather(rows[sl], [eidx, col])
                        plsc.store_scatter(rows[sl], [eidx, col], tv * ev)
                    return carry2
                lax.fori_loop(_c(0), _c(CH // 8), sbody, _c(0))
            pltpu.sync_copy(rows[sl], spmem.at[idxb[sl]], add=True)
        return carry
    lax.fori_loop(_c(0), _c(NBLOCK), block, _c(0))

    plsc.subcore_barrier()
    _dump_spmem(spmem, rows0, ypart_hbm, c, sid)


_scagg = pl.kernel(
    _scagg_body,
    out_type=[jax.ShapeDtypeStruct((NC, NSEG, CH), _f32)],
    mesh=_MESH,
    compiler_params=pltpu.CompilerParams(needs_layout_passes=False),
    scratch_types=(
        [pltpu.VMEM((BLKE,), _i32)] * 2
        + [pltpu.VMEM((BLKE,), _f32)]
        + [pltpu.VMEM((K,), _i32)] * 4
        + [pltpu.VMEM((K, CH), _f32)] * 4
        + [pltpu.VMEM_SHARED((NSEG, CH), _f32)]
        + [pltpu.SemaphoreType.DMA] * 4
    ),
)


_RB = 1280


def _z(v=0):
    return jnp.array(v, _i32)


def _tc_prep_body(x_ref, rel_ref, a_ref):
    r = pl.program_id(1)
    a_ref[...] = x_ref[...] * rel_ref[pl.ds(r, 1), :]


_tc_prep = pl.pallas_call(
    _tc_prep_body,
    grid=(NSEG // _RB, NRELROW),
    in_specs=[pl.BlockSpec((_RB, CH), lambda b, r: (b, _z())),
              pl.BlockSpec((NRELROW, CH), lambda b, r: (_z(), _z())),],
    out_specs=pl.BlockSpec((_RB, CH), lambda b, r: (r * _z(NSEG // _RB) + b, _z())),
    out_shape=jax.ShapeDtypeStruct((NRELROW * NSEG, CH), _f32),
)


def _norm_rows(a):
    ss = jnp.sum(a * a, axis=1, keepdims=True)
    return a * lax.rsqrt(jnp.maximum(ss, 1e-24))


def _tc_merge_prep_body(pp_ref, rel_ref, x_ref, a_ref):
    r = pl.program_id(1)
    y = _norm_rows(pp_ref[0] + pp_ref[1])
    x_ref[...] = y
    a_ref[...] = y * rel_ref[pl.ds(r, 1), :]


_tc_merge_prep = pl.pallas_call(
    _tc_merge_prep_body,
    grid=(NSEG // _RB, NRELROW),
    in_specs=[pl.BlockSpec((NC, _RB, CH), lambda b, r: (_z(), b, _z())),
              pl.BlockSpec((NRELROW, CH), lambda b, r: (_z(), _z())),],
    out_specs=[pl.BlockSpec((_RB, CH), lambda b, r: (b, _z())),
               pl.BlockSpec((_RB, CH), lambda b, r: (r * _z(NSEG // _RB) + b, _z()))],
    out_shape=[jax.ShapeDtypeStruct((NSEG, CH), _f32),
               jax.ShapeDtypeStruct((NRELROW * NSEG, CH), _f32)],
)


def _tc_merge_body(pp_ref, x_ref):
    x_ref[...] = _norm_rows(pp_ref[0] + pp_ref[1])


_tc_merge = pl.pallas_call(
    _tc_merge_body,
    grid=(NSEG // _RB,),
    in_specs=[pl.BlockSpec((NC, _RB, CH), lambda b: (_z(), b, _z()))],
    out_specs=pl.BlockSpec((_RB, CH), lambda b: (b, _z())),
    out_shape=jax.ShapeDtypeStruct((NSEG, CH), _f32),
)


_EXTRA = EPAD - NEDGE


def kernel(user_emb, item_emb, edge_index, edge_type, inter_edge,
           inter_edge_w, relation_emb):
    del user_emb
    head = jnp.pad(edge_index[0].astype(_i32), (0, _EXTRA),
                   constant_values=NSEG - 1)
    tail = jnp.pad(edge_index[1].astype(_i32), (0, _EXTRA))
    rel = jnp.pad(edge_type.astype(_i32), (0, _EXTRA), constant_values=1)
    src = jnp.pad(inter_edge[0].astype(_i32), (0, _EXTRA),
                  constant_values=NSEG - 1)
    dst = jnp.pad(inter_edge[1].astype(_i32), (0, _EXTRA))
    iw = jnp.pad(inter_edge_w.astype(_f32), (0, _EXTRA))
    relemb = relation_emb.astype(_f32)

    x = jnp.pad(item_emb.astype(_f32), ((0, NSEG - NENT), (0, 0)))
    a = _tc_prep(x, relemb)
    for hop in range(2):
        scores, maxpart = _sca(a, x, head, rel, tail)
        (evals,) = _scm(head, scores, maxpart)
        (ypart,) = _scagg(x, head, tail, evals)
        if hop == 0:
            x, a = _tc_merge_prep(ypart, relemb)
        else:
            x = _tc_merge(ypart)
    (upart,) = _scagg(x, src, dst, iw)
    user_out = _tc_merge(upart)
    return user_out[:NENT], x[:NENT]


# speedup vs baseline: 1.0014x; 1.0014x over previous
import jax
import jax.numpy as jnp
from jax import lax
from jax.experimental import pallas as pl
from jax.experimental.pallas import tpu as pltpu
from jax.experimental.pallas import tpu_sc as plsc

NENT = 10000
NSEG = 10240
CH = 128
NEDGE = 320000
NRELROW = 9
NC = 2
NS = 16
NW = NC * NS
EPW = 10240
EPAD = NW * EPW
K = 80
KH = K // 2
NCHUNK = EPW // K
NG = K // 16
BLK = 8
BLKE = BLK * K
NBLOCK = NCHUNK // BLK
DEPTH = 4
ROWS_PER_TILE = NSEG // NS
MBLK = 128
MROUNDS = NSEG // MBLK

_MESH = plsc.VectorSubcoreMesh(core_axis_name="c", subcore_axis_name="s")
_f32 = jnp.float32
_i32 = jnp.int32


def _c(v):
    return jnp.array(v, _i32)


def _worker_id():
    return lax.axis_index("s") * _c(NC) + lax.axis_index("c")


def _retry_scatter_max(tab, hidx, sv):
    def bdy(go):
        cur = plsc.load_gather(tab, [hidx])
        plsc.store_scatter(tab, [hidx], jnp.maximum(sv, cur), mask=sv > cur)
        chk = plsc.load_gather(tab, [hidx])
        return jnp.max((sv > chk).astype(_i32))
    lax.while_loop(lambda go: go > _c(0), bdy, _c(1))


def _gather_rows_split(table_hbm, idxb, rows, sem):
    d0 = pltpu.async_copy(table_hbm.at[idxb.at[pl.ds(0, KH)]],
                          rows.at[pl.ds(0, KH)], sem)
    d1 = pltpu.async_copy(table_hbm.at[idxb.at[pl.ds(KH, KH)]],
                          rows.at[pl.ds(KH, KH)], sem)
    return (d0, d1)


def _sca_body(a_hbm, x_hbm, head_hbm, rel_hbm, tail_hbm,
              scores_hbm, maxpart_hbm,
              headblk, relblk, tailblk, sball, maxtab,
              arows0, arows1, arows2, arows3,
              trows0, trows1, trows2, trows3,
              cidxb0, cidxb1, cidxb2, cidxb3,
              tailb0, tailb1, tailb2, tailb3,
              sem0, sem1, sem2, sem3):
    w = _worker_id()
    base0 = w * _c(EPW)
    lanes = lax.iota(_i32, 16)
    zero16 = jnp.zeros((16,), _f32)
    zero16i = jnp.zeros((16,), _i32)

    def initb(i, carry):
        plsc.store_scatter(maxtab, [lanes + i * _c(16)], zero16)
        return carry
    lax.fori_loop(_c(0), _c(NSEG // 16), initb, _c(0))

    arows = (arows0, arows1, arows2, arows3)
    trows = (trows0, trows1, trows2, trows3)
    cidxb = (cidxb0, cidxb1, cidxb2, cidxb3)
    tailb = (tailb0, tailb1, tailb2, tailb3)
    sems = (sem0, sem1, sem2, sem3)

    def issue(j, slot):
        for g in range(NG):
            off = j * K + g * 16
            h16 = headblk[pl.ds(off, 16)]
            r16 = relblk[pl.ds(off, 16)]
            rid = jnp.where(r16 == _c(0), _c(NRELROW - 1), r16 - _c(1))
            cidxb[slot][pl.ds(g * 16, 16)] = rid * _c(NSEG) + h16
            tailb[slot][pl.ds(g * 16, 16)] = tailblk[pl.ds(off, 16)]
        da = _gather_rows_split(a_hbm, cidxb[slot], arows[slot], sems[slot])
        dt = _gather_rows_split(x_hbm, tailb[slot], trows[slot], sems[slot])
        return da + dt

    def block(b, carry):
        eb0 = base0 + b * _c(BLKE)
        pltpu.sync_copy(head_hbm.at[pl.ds(eb0, BLKE)], headblk)
        pltpu.sync_copy(rel_hbm.at[pl.ds(eb0, BLKE)], relblk)
        pltpu.sync_copy(tail_hbm.at[pl.ds(eb0, BLKE)], tailblk)
        ds_ = [None] * BLK
        for p in range(DEPTH):
            ds_[p] = issue(p, p)
        for j in range(BLK):
            sl = j % DEPTH
            for d in ds_[j]:
                d.wait()
            for g in range(NG):
                eidx = lanes + _c(g * 16)

                def cbody(i2, accs, sl=sl, eidx=eidx):
                    ch0 = i2 * _c(8)
                    outs = list(accs)
                    for k in range(8):
                        col = zero16i + (ch0 + _c(k))
                        av = plsc.load_gather(arows[sl], [eidx, col])
                        tv = plsc.load_gather(trows[sl], [eidx, col])
                        outs[k % 4] = outs[k % 4] + av * tv
                    return tuple(outs)
                accs = lax.fori_loop(_c(0), _c(CH // 8), cbody,
                                     (zero16, zero16, zero16, zero16))
                dot = (accs[0] + accs[1]) + (accs[2] + accs[3])
                sv = jnp.exp(dot)
                gidx = lanes + b * _c(BLKE) + _c(j * K + g * 16)
                plsc.store_scatter(sball, [gidx], sv)
                hidx = headblk[pl.ds(j * K + g * 16, 16)]
                _retry_scatter_max(maxtab, hidx, sv)
            if j + DEPTH < BLK:
                ds_[j + DEPTH] = issue(j + DEPTH, sl)
        return carry
    lax.fori_loop(_c(0), _c(NBLOCK), block, _c(0))

    pltpu.sync_copy(sball, scores_hbm.at[pl.ds(base0, EPW)])
    pltpu.sync_copy(maxtab, maxpart_hbm.at[w])


_sca = pl.kernel(
    _sca_body,
    out_type=[jax.ShapeDtypeStruct((EPAD,), _f32),
              jax.ShapeDtypeStruct((NW, NSEG), _f32)],
    mesh=_MESH,
    compiler_params=pltpu.CompilerParams(needs_layout_passes=False),
    scratch_types=(
        [pltpu.VMEM((BLKE,), _i32)] * 3
        + [pltpu.VMEM((EPW,), _f32),
           pltpu.VMEM((NSEG,), _f32)]
        + [pltpu.VMEM((K, CH), _f32)] * 8
        + [pltpu.VMEM((K,), _i32)] * 8
        + [pltpu.SemaphoreType.DMA] * 4
    ),
)


def _scm_body(head_hbm, scores_hbm, maxpart_hbm,
              evals_hbm,
              headall, sball, eall, mtab, mstage):
    w = _worker_id()
    base0 = w * _c(EPW)
    lanes = lax.iota(_i32, 16)

    pltpu.sync_copy(head_hbm.at[pl.ds(base0, EPW)], headall)
    pltpu.sync_copy(scores_hbm.at[pl.ds(base0, EPW)], sball)

    def mround(r, carry):
        seg0 = r * _c(MBLK)
        pltpu.sync_copy(maxpart_hbm.at[:, pl.ds(seg0, MBLK)], mstage)
        for g in range(MBLK // 16):
            idx = lanes + _c(g * 16)
            m = plsc.load_gather(mstage, [jnp.full((16,), 0, _i32), idx])
            for j in range(1, NW):
                vj = plsc.load_gather(mstage, [jnp.full((16,), j, _i32), idx])
                m = jnp.maximum(m, vj)
            plsc.store_scatter(mtab, [idx + seg0], m)
        return carry
    lax.fori_loop(_c(0), _c(MROUNDS), mround, _c(0))

    def ebody(i, carry):
        idx = lanes + i * _c(16)
        h16 = plsc.load_gather(headall, [idx])
        m16 = plsc.load_gather(mtab, [h16])
        s16 = plsc.load_gather(sball, [idx])
        plsc.store_scatter(eall, [idx], jnp.exp(s16 - m16))
        return carry
    lax.fori_loop(_c(0), _c(EPW // 16), ebody, _c(0))

    pltpu.sync_copy(eall, evals_hbm.at[pl.ds(base0, EPW)])


_scm = pl.kernel(
    _scm_body,
    out_type=[jax.ShapeDtypeStruct((EPAD,), _f32)],
    mesh=_MESH,
    compiler_params=pltpu.CompilerParams(needs_layout_passes=False),
    scratch_types=[
        pltpu.VMEM((EPW,), _i32),
        pltpu.VMEM((EPW,), _f32),
        pltpu.VMEM((EPW,), _f32),
        pltpu.VMEM((NSEG,), _f32),
        pltpu.VMEM((NW, MBLK), _f32),
    ],
)


def _zero_rows_buf(buf):
    lanes = lax.iota(_i32, 16)
    zero16 = jnp.zeros((16,), _f32)

    def zb(i, carry):
        flat = lanes + i * _c(16)
        plsc.store_scatter(buf, [lax.shift_right_logical(flat, _c(7)),
                                 lax.bitwise_and(flat, _c(127))], zero16)
        return carry
    lax.fori_loop(_c(0), _c(K * CH // 16), zb, _c(0))


def _zero_spmem_slice(spmem, buf, sid):
    def zloop(j, carry):
        pltpu.sync_copy(
            buf, spmem.at[pl.ds(sid * _c(ROWS_PER_TILE) + j * _c(K), K)])
        return carry
    lax.fori_loop(_c(0), _c(ROWS_PER_TILE // K), zloop, _c(0))


def _dump_spmem(spmem, buf, ypart_hbm, c, sid):
    for j in range(ROWS_PER_TILE // K):
        row = sid * _c(ROWS_PER_TILE) + _c(j * K)
        pltpu.sync_copy(spmem.at[pl.ds(row, K)], buf)
        pltpu.sync_copy(buf, ypart_hbm.at[c, pl.ds(row, K)])


def _scagg_body(x_hbm, idx_hbm, tail_hbm, ev_hbm,
                ypart_hbm,
                idxiblk, idxtblk, evblk,
                idxb0, idxb1, idxb2, idxb3,
                rows0, rows1, rows2, rows3,
                spmem, sem0, sem1, sem2, sem3):
    c = lax.axis_index("c")
    sid = lax.axis_index("s")
    w = _worker_id()
    base0 = w * _c(EPW)
    lanes = lax.iota(_i32, 16)

    _zero_rows_buf(rows0)
    _zero_spmem_slice(spmem, rows0, sid)
    plsc.subcore_barrier()

    rows = (rows0, rows1, rows2, rows3)
    idxb = (idxb0, idxb1, idxb2, idxb3)
    sems = (sem0, sem1, sem2, sem3)

    def block(b, carry):
        eb0 = base0 + b * _c(BLKE)
        pltpu.sync_copy(idx_hbm.at[pl.ds(eb0, BLKE)], idxiblk)
        pltpu.sync_copy(tail_hbm.at[pl.ds(eb0, BLKE)], idxtblk)
        pltpu.sync_copy(ev_hbm.at[pl.ds(eb0, BLKE)], evblk)
        ds_ = [None] * BLK

        def issue(j, slot):
            return _gather_rows_split(
                x_hbm, idxtblk.at[pl.ds(j * K, K)], rows[slot], sems[slot])
        for p in range(DEPTH):
            ds_[p] = issue(p, p)
        for j in range(BLK):
            sl = j % DEPTH
            for d in ds_[j]:
                d.wait()
            for g in range(NG):
                iv = idxiblk[pl.ds(j * K + g * 16, 16)]
                idxb[sl][pl.ds(g * 16, 16)] = iv
                ev = evblk[pl.ds(j * K + g * 16, 16)]
                eidx = lanes + _c(g * 16)

                def sbody(i2, carry2, sl=sl, eidx=eidx, ev=ev):
                    ch0 = i2 * _c(8)
                    for k in range(8):
                        col = jnp.zeros((16,), _i32) + (ch0 + _c(k))
                        tv = plsc.load_gather(rows[sl], [eidx, col])
                        plsc.store_scatter(rows[sl], [eidx, col], tv * ev)
                    return carry2
                lax.fori_loop(_c(0), _c(CH // 8), sbody, _c(0))
            pltpu.sync_copy(rows[sl], spmem.at[idxb[sl]], add=True)
            if j + DEPTH < BLK:
                ds_[j + DEPTH] = issue(j + DEPTH, sl)
        return carry
    lax.fori_loop(_c(0), _c(NBLOCK), block, _c(0))

    plsc.subcore_barrier()
    _dump_spmem(spmem, rows0, ypart_hbm, c, sid)


_scagg = pl.kernel(
    _scagg_body,
    out_type=[jax.ShapeDtypeStruct((NC, NSEG, CH), _f32)],
    mesh=_MESH,
    compiler_params=pltpu.CompilerParams(needs_layout_passes=False),
    scratch_types=(
        [pltpu.VMEM((BLKE,), _i32)] * 2
        + [pltpu.VMEM((BLKE,), _f32)]
        + [pltpu.VMEM((K,), _i32)] * 4
        + [pltpu.VMEM((K, CH), _f32)] * 4
        + [pltpu.VMEM_SHARED((NSEG, CH), _f32)]
        + [pltpu.SemaphoreType.DMA] * 4
    ),
)


_RB = 1280


def _z(v=0):
    return jnp.array(v, _i32)


def _tc_prep_body(x_ref, rel_ref, a_ref):
    r = pl.program_id(1)
    a_ref[...] = x_ref[...] * rel_ref[pl.ds(r, 1), :]


_tc_prep = pl.pallas_call(
    _tc_prep_body,
    grid=(NSEG // _RB, NRELROW),
    in_specs=[pl.BlockSpec((_RB, CH), lambda b, r: (b, _z())),
              pl.BlockSpec((NRELROW, CH), lambda b, r: (_z(), _z())),],
    out_specs=pl.BlockSpec((_RB, CH), lambda b, r: (r * _z(NSEG // _RB) + b, _z())),
    out_shape=jax.ShapeDtypeStruct((NRELROW * NSEG, CH), _f32),
)


def _norm_rows(a):
    ss = jnp.sum(a * a, axis=1, keepdims=True)
    return a * lax.rsqrt(jnp.maximum(ss, 1e-24))


def _tc_merge_prep_body(pp_ref, rel_ref, x_ref, a_ref):
    r = pl.program_id(1)
    y = _norm_rows(pp_ref[0] + pp_ref[1])
    x_ref[...] = y
    a_ref[...] = y * rel_ref[pl.ds(r, 1), :]


_tc_merge_prep = pl.pallas_call(
    _tc_merge_prep_body,
    grid=(NSEG // _RB, NRELROW),
    in_specs=[pl.BlockSpec((NC, _RB, CH), lambda b, r: (_z(), b, _z())),
              pl.BlockSpec((NRELROW, CH), lambda b, r: (_z(), _z())),],
    out_specs=[pl.BlockSpec((_RB, CH), lambda b, r: (b, _z())),
               pl.BlockSpec((_RB, CH), lambda b, r: (r * _z(NSEG // _RB) + b, _z()))],
    out_shape=[jax.ShapeDtypeStruct((NSEG, CH), _f32),
               jax.ShapeDtypeStruct((NRELROW * NSEG, CH), _f32)],
)


def _tc_merge_body(pp_ref, x_ref):
    x_ref[...] = _norm_rows(pp_ref[0] + pp_ref[1])


_tc_merge = pl.pallas_call(
    _tc_merge_body,
    grid=(NSEG // _RB,),
    in_specs=[pl.BlockSpec((NC, _RB, CH), lambda b: (_z(), b, _z()))],
    out_specs=pl.BlockSpec((_RB, CH), lambda b: (b, _z())),
    out_shape=jax.ShapeDtypeStruct((NSEG, CH), _f32),
)


_EXTRA = EPAD - NEDGE


def kernel(user_emb, item_emb, edge_index, edge_type, inter_edge,
           inter_edge_w, relation_emb):
    del user_emb
    head = jnp.pad(edge_index[0].astype(_i32), (0, _EXTRA),
                   constant_values=NSEG - 1)
    tail = jnp.pad(edge_index[1].astype(_i32), (0, _EXTRA))
    rel = jnp.pad(edge_type.astype(_i32), (0, _EXTRA), constant_values=1)
    src = jnp.pad(inter_edge[0].astype(_i32), (0, _EXTRA),
                  constant_values=NSEG - 1)
    dst = jnp.pad(inter_edge[1].astype(_i32), (0, _EXTRA))
    iw = jnp.pad(inter_edge_w.astype(_f32), (0, _EXTRA))
    relemb = relation_emb.astype(_f32)

    x = jnp.pad(item_emb.astype(_f32), ((0, NSEG - NENT), (0, 0)))
    a = _tc_prep(x, relemb)
    for hop in range(2):
        scores, maxpart = _sca(a, x, head, rel, tail)
        (evals,) = _scm(head, scores, maxpart)
        (ypart,) = _scagg(x, head, tail, evals)
        if hop == 0:
            x, a = _tc_merge_prep(ypart, relemb)
        else:
            x = _tc_merge(ypart)
    (upart,) = _scagg(x, src, dst, iw)
    user_out = _tc_merge(upart)
    return user_out[:NENT], x[:NENT]
